# baseline (speedup 1.0000x reference)
import jax
import jax.numpy as jnp
from jax.experimental import pallas as pl
from jax.experimental.pallas import tpu as pltpu

_N = 33554432
_COLS = 1024
_ROWS = _N // _COLS
_BLOCK_ROWS = 512
_GRID = _ROWS // _BLOCK_ROWS


def _copy_kernel(x_ref, o_ref):
    o_ref[...] = x_ref[...]

    @pl.when(pl.program_id(0) == 0)
    def _():
        col = jax.lax.broadcasted_iota(jnp.int32, (1, _COLS), 1)
        o_ref[0:1, :] = jnp.where(col == 0, 0.0, x_ref[0:1, :])


def kernel(x):
    x2 = x.reshape(_ROWS, _COLS)
    out = pl.pallas_call(
        _copy_kernel,
        out_shape=jax.ShapeDtypeStruct((_ROWS, _COLS), x.dtype),
        grid=(_GRID,),
        in_specs=[pl.BlockSpec((_BLOCK_ROWS, _COLS), lambda i: (i, 0))],
        out_specs=pl.BlockSpec((_BLOCK_ROWS, _COLS), lambda i: (i, 0)),
    )(x2)
    return out.reshape(_N)
